# baseline (speedup 1.0000x reference)
import jax
import jax.numpy as jnp
from jax.experimental import pallas as pl

_R = 512


def _logsoftmax_kernel(x_ref, w_ref, b_ref, o_ref):
    logits = (
        jnp.dot(w_ref[...], x_ref[0], preferred_element_type=jnp.float32)
        + b_ref[...]
    )
    m = jnp.max(logits, axis=0, keepdims=True)
    lse = jnp.log(jnp.sum(jnp.exp(logits - m), axis=0, keepdims=True))
    o_ref[0] = logits - m - lse


@jax.jit
def kernel(x, xl, W, b):
    B, T, D = x.shape
    V = W.shape[0]
    xt = x.transpose(0, 2, 1).astype(jnp.bfloat16)
    wb = W.astype(jnp.bfloat16)
    b2 = b.reshape(V, 1)

    out_t = pl.pallas_call(
        _logsoftmax_kernel,
        grid=(B, T // _R),
        in_specs=[
            pl.BlockSpec((1, D, _R), lambda bi, ti: (bi, 0, ti)),
            pl.BlockSpec((V, D), lambda bi, ti: (0, 0)),
            pl.BlockSpec((V, 1), lambda bi, ti: (0, 0)),
        ],
        out_specs=pl.BlockSpec((1, V, _R), lambda bi, ti: (bi, 0, ti)),
        out_shape=jax.ShapeDtypeStruct((B, V, T), jnp.float32),
    )(xt, wb, b2)
    return out_t.transpose(0, 2, 1)


# speedup vs baseline: 1.2178x; 1.2178x over previous
import jax
import jax.numpy as jnp
from jax.experimental import pallas as pl

_R = 256
_LOG2E = 1.4426950408889634
_LN2 = 0.6931471805599453


def _logsoftmax_kernel(x_ref, w_ref, b_ref, ab_ref, o_ref):
    xb = x_ref[0]
    l2 = (
        jnp.dot(w_ref[...], xb, preferred_element_type=jnp.float32)
        + b_ref[...]
    )
    xf = xb.astype(jnp.float32)
    n2 = jnp.sum(xf * xf, axis=0, keepdims=True)
    c = ab_ref[0, 0] * jnp.sqrt(n2) + ab_ref[0, 1]
    s = jnp.sum(jnp.exp2(l2 - c), axis=0, keepdims=True)
    off = (c + jnp.log2(s)) * _LN2
    o_ref[0] = l2 * _LN2 - off


@jax.jit
def kernel(x, xl, W, b):
    B, T, D = x.shape
    V = W.shape[0]
    xt = x.transpose(0, 2, 1).astype(jnp.bfloat16)
    wb = (W * _LOG2E).astype(jnp.bfloat16)
    b2 = (b * _LOG2E).reshape(V, 1).astype(jnp.float32)
    wf = wb.astype(jnp.float32)
    a_max = jnp.max(jnp.sqrt(jnp.sum(wf * wf, axis=1)))
    b_max = jnp.max(b2) + 0.5
    ab = jnp.stack([a_max, b_max]).reshape(1, 2)

    out_t = pl.pallas_call(
        _logsoftmax_kernel,
        grid=(B, T // _R),
        in_specs=[
            pl.BlockSpec((1, D, _R), lambda bi, ti: (bi, 0, ti)),
            pl.BlockSpec((V, D), lambda bi, ti: (0, 0)),
            pl.BlockSpec((V, 1), lambda bi, ti: (0, 0)),
            pl.BlockSpec((1, 2), lambda bi, ti: (0, 0)),
        ],
        out_specs=pl.BlockSpec((1, V, _R), lambda bi, ti: (bi, 0, ti)),
        out_shape=jax.ShapeDtypeStruct((B, V, T), jnp.float32),
    )(xt, wb, b2, ab)
    return out_t.transpose(0, 2, 1)


# speedup vs baseline: 1.2610x; 1.0355x over previous
import jax
import jax.numpy as jnp
from jax.experimental import pallas as pl

_R = 256
_KA = 136
_LOG2E = 1.4426950408889634
_LN2 = 0.6931471805599453


def _logsoftmax_kernel(x_ref, w_ref, a_ref, o_ref):
    xb = x_ref[0]
    l2 = jnp.dot(w_ref[...], xb, preferred_element_type=jnp.float32)
    xf = xb.astype(jnp.float32)
    n2 = jnp.sum(xf * xf, axis=0, keepdims=True)
    c = a_ref[0, 0] * jnp.sqrt(n2) + 0.5
    s = jnp.sum(jnp.exp2(l2 - c), axis=0, keepdims=True)
    noff = (c + jnp.log2(s)) * (-_LN2)
    o_ref[0] = l2 * _LN2 + noff


@jax.jit
def kernel(x, xl, W, b):
    B, T, D = x.shape
    V = W.shape[0]
    xb16 = x.astype(jnp.bfloat16).transpose(0, 2, 1)
    ones = jnp.ones((B, 1, T), dtype=jnp.bfloat16)
    zeros = jnp.zeros((B, _KA - D - 1, T), dtype=jnp.bfloat16)
    xt = jnp.concatenate([xb16, ones, zeros], axis=1)
    wb16 = (W * _LOG2E).astype(jnp.bfloat16)
    bcol = (b * _LOG2E).astype(jnp.bfloat16).reshape(V, 1)
    wzeros = jnp.zeros((V, _KA - D - 1), dtype=jnp.bfloat16)
    wa = jnp.concatenate([wb16, bcol, wzeros], axis=1)
    wf = wa.astype(jnp.float32)
    a_max = jnp.max(jnp.sqrt(jnp.sum(wf * wf, axis=1))).reshape(1, 1)

    out_t = pl.pallas_call(
        _logsoftmax_kernel,
        grid=(B, T // _R),
        in_specs=[
            pl.BlockSpec((1, _KA, _R), lambda bi, ti: (bi, 0, ti)),
            pl.BlockSpec((V, _KA), lambda bi, ti: (0, 0)),
            pl.BlockSpec((1, 1), lambda bi, ti: (0, 0)),
        ],
        out_specs=pl.BlockSpec((1, V, _R), lambda bi, ti: (bi, 0, ti)),
        out_shape=jax.ShapeDtypeStruct((B, V, T), jnp.float32),
    )(xt, wa, a_max)
    return out_t.transpose(0, 2, 1)


# speedup vs baseline: 1.3878x; 1.1005x over previous
import jax
import jax.numpy as jnp
from jax.experimental import pallas as pl

_R = 512
_KA = 136
_LOG2E = 1.4426950408889634
_LN2 = 0.6931471805599453


def _logsoftmax_kernel(x_ref, w_ref, a_ref, o_ref):
    xb = x_ref[0]
    l2 = jnp.dot(w_ref[...], xb, preferred_element_type=jnp.float32)
    xf = xb.astype(jnp.float32)
    n2 = jnp.sum(xf * xf, axis=0, keepdims=True)
    c = a_ref[0, 0] * jnp.sqrt(n2) + 0.5
    s = jnp.sum(jnp.exp2(l2 - c), axis=0, keepdims=True)
    noff = (c + jnp.log2(s)) * (-_LN2)
    o_ref[0] = l2 * _LN2 + noff


@jax.jit
def kernel(x, xl, W, b):
    B, T, D = x.shape
    V = W.shape[0]
    xb16 = x.astype(jnp.bfloat16).transpose(0, 2, 1)
    ones = jnp.ones((B, 1, T), dtype=jnp.bfloat16)
    zeros = jnp.zeros((B, _KA - D - 1, T), dtype=jnp.bfloat16)
    xt = jnp.concatenate([xb16, ones, zeros], axis=1)
    wb16 = (W * _LOG2E).astype(jnp.bfloat16)
    bcol = (b * _LOG2E).astype(jnp.bfloat16).reshape(V, 1)
    wzeros = jnp.zeros((V, _KA - D - 1), dtype=jnp.bfloat16)
    wa = jnp.concatenate([wb16, bcol, wzeros], axis=1)
    wf = wa.astype(jnp.float32)
    a_max = jnp.max(jnp.sqrt(jnp.sum(wf * wf, axis=1))).reshape(1, 1)

    out_t = pl.pallas_call(
        _logsoftmax_kernel,
        grid=(B, T // _R),
        in_specs=[
            pl.BlockSpec((1, _KA, _R), lambda bi, ti: (bi, 0, ti)),
            pl.BlockSpec((V, _KA), lambda bi, ti: (0, 0)),
            pl.BlockSpec((1, 1), lambda bi, ti: (0, 0)),
        ],
        out_specs=pl.BlockSpec((1, V, _R), lambda bi, ti: (bi, 0, ti)),
        out_shape=jax.ShapeDtypeStruct((B, V, T), jnp.float32),
    )(xt, wa, a_max)
    return out_t.transpose(0, 2, 1)
